# Initial kernel scaffold; baseline (speedup 1.0000x reference)
import jax
import jax.numpy as jnp
from jax.experimental import pallas as pl


def kernel(node_feat, edge_attri, edge_index, W1, b1, W2, b2):
    raise NotImplementedError("write your pallas kernel here")


# speedup vs baseline: 3.7315x; 3.7315x over previous
import functools

import jax
import jax.numpy as jnp
from jax import lax
from jax.experimental import pallas as pl
from jax.experimental.pallas import tpu as pltpu
from jax.experimental.pallas import tpu_sc as plsc

_N = 50000
_E = 1600000
_NIN = 24

_NC = 2
_NS = 16
_NW = _NC * _NS

_CHUNK = 2000
_ROWS = _E // _CHUNK
_ROWS_PER_W = _ROWS // _NW

_NB = 1000
_EB = 5000


def _mlp_body(x_ref, w1_ref, b1_ref, w2_ref, b2_ref, o_ref):
    z = jnp.dot(x_ref[...], w1_ref[...], preferred_element_type=jnp.float32)
    z = z + b1_ref[...]
    h = z * (1.0 / (1.0 + jnp.exp(-z)))
    o_ref[...] = jnp.dot(h, w2_ref[...], preferred_element_type=jnp.float32) + b2_ref[...]


def _node_mlp(x2d, W1, b1, W2, b2):
    grid = (_N // _NB,)
    return pl.pallas_call(
        _mlp_body,
        grid=grid,
        in_specs=[
            pl.BlockSpec((_NB, _NIN), lambda i: (i, 0)),
            pl.BlockSpec((_NIN, 128), lambda i: (0, 0)),
            pl.BlockSpec((1, 128), lambda i: (0, 0)),
            pl.BlockSpec((128, 1), lambda i: (0, 0)),
            pl.BlockSpec((1, 1), lambda i: (0, 0)),
        ],
        out_specs=pl.BlockSpec((_NB, 1), lambda i: (i, 0)),
        out_shape=jax.ShapeDtypeStruct((_N, 1), jnp.float32),
    )(x2d, W1, b1.reshape(1, 128), W2, b2.reshape(1, 1))


def _gather_body(nw_hbm, idx_hbm, out_hbm, table_v, idx_v, out_v):
    wid = lax.axis_index("s") * _NC + lax.axis_index("c")
    pltpu.sync_copy(nw_hbm, table_v)

    def do_row(c, carry):
        r = wid * _ROWS_PER_W + c
        pltpu.sync_copy(idx_hbm.at[0, r], idx_v)

        def do_vreg(j, carry2):
            iv = idx_v[pl.ds(j * 16, 16)]
            out_v[pl.ds(j * 16, 16)] = plsc.load_gather(table_v, [iv])
            return carry2

        lax.fori_loop(0, _CHUNK // 16, do_vreg, 0)
        pltpu.sync_copy(out_v, out_hbm.at[r])
        return carry

    lax.fori_loop(0, _ROWS_PER_W, do_row, 0)


def _edge_gather(nw_flat, edge_idx3):
    mesh = plsc.VectorSubcoreMesh(core_axis_name="c", subcore_axis_name="s")
    call = pl.kernel(
        _gather_body,
        out_type=jax.ShapeDtypeStruct((_ROWS, _CHUNK), jnp.float32),
        mesh=mesh,
        scratch_types=[
            pltpu.VMEM((_N,), jnp.float32),
            pltpu.VMEM((_CHUNK,), jnp.int32),
            pltpu.VMEM((_CHUNK,), jnp.float32),
        ],
        compiler_params=pltpu.CompilerParams(needs_layout_passes=False),
    )
    return call(nw_flat, edge_idx3)


def _mul_body(a_ref, w_ref, o_ref):
    o_ref[...] = a_ref[...] * w_ref[...]


def _edge_mul(attr2d, ew2d):
    grid = (_E // _EB,)
    return pl.pallas_call(
        _mul_body,
        grid=grid,
        in_specs=[
            pl.BlockSpec((_EB, _NIN), lambda i: (i, 0)),
            pl.BlockSpec((_EB, 1), lambda i: (i, 0)),
        ],
        out_specs=pl.BlockSpec((_EB, _NIN), lambda i: (i, 0)),
        out_shape=jax.ShapeDtypeStruct((_E, _NIN), jnp.float32),
    )(attr2d, ew2d)


def kernel(node_feat, edge_attri, edge_index, W1, b1, W2, b2):
    x2d = node_feat.reshape(_N, _NIN)
    nw = _node_mlp(x2d, W1, b1, W2, b2)
    ew = _edge_gather(nw.reshape(_N), edge_index.reshape(2, _ROWS, _CHUNK))
    attr2d = edge_attri.reshape(_E, _NIN)
    out2d = _edge_mul(attr2d, ew.reshape(_E, 1))
    return out2d.reshape(_E, 4, 3, 2)
